# baseline (speedup 1.0000x reference)
import functools
import math

import jax
import jax.numpy as jnp
from jax import lax
from jax.experimental import pallas as pl
from jax.experimental.pallas import tpu as pltpu
from jax.experimental.pallas import tpu_sc as plsc

NC = 2
NS = 16
NW = NC * NS
LANES = 16
RW = 240
G = 32
DEPTH = 2


def _make_kernel(B, V, D):
    per_w = B // NW
    NG = per_w // G
    SUPER = NG // DEPTH
    n_full = D // LANES
    rem = D - n_full * LANES
    tail_base = D - LANES
    VR = (V * D) // RW
    max_s = RW - math.gcd(D, RW)
    assert V * D == VR * RW and max_s + D <= 2 * RW and (RW * 4) % 64 == 0

    mesh = plsc.VectorSubcoreMesh(core_axis_name="c", subcore_axis_name="s")

    @functools.partial(
        pl.kernel,
        mesh=mesh,
        compiler_params=pltpu.CompilerParams(
            use_tc_tiling_on_sc=False, needs_layout_passes=False),
        out_type=jax.ShapeDtypeStruct((B,), jnp.float32),
        scratch_types=[
            pltpu.VMEM((per_w, 2), jnp.int32),
            pltpu.VMEM((2 * per_w,), jnp.int32),
            pltpu.VMEM((2 * per_w,), jnp.int32),
            pltpu.VMEM((per_w,), jnp.int32),
            pltpu.VMEM((per_w,), jnp.int32),
            pltpu.VMEM((DEPTH, 2 * G, RW), jnp.float32),
            pltpu.VMEM((DEPTH, 2 * G, RW), jnp.float32),
            pltpu.VMEM((per_w,), jnp.float32),
            pltpu.SemaphoreType.DMA,
            pltpu.SemaphoreType.DMA,
            pltpu.SemaphoreType.DMA,
            pltpu.SemaphoreType.DMA,
        ],
    )
    def k(x_hbm, ine_hbm, oute_hbm, out_hbm,
          xv, ixi, ixo, svi, svo, rin, rout, res_v, si0, so0, si1, so1):
        wid = lax.axis_index("s") * NC + lax.axis_index("c")
        base = wid * per_w
        lane = lax.iota(jnp.int32, LANES)
        zero16 = lane * 0
        one16 = zero16 + 1
        tail_mask = lane >= (LANES - rem)
        perms = [lane ^ kk for kk in (8, 4, 2, 1)]
        sems = [(si0, so0), (si1, so1)]

        pltpu.sync_copy(x_hbm.at[pl.ds(base, per_w), :], xv)

        def build(cc, carry):
            q = cc * LANES + lane
            for col, ix, sv in ((zero16, ixi, svi), (one16, ixo, svo)):
                j = plsc.load_gather(xv, [q, col])
                flat = j * D
                r0 = flat // RW
                plsc.store_scatter(ix, [q * 2], r0)
                plsc.store_scatter(ix, [q * 2 + 1], r0 + 1)
                plsc.store_scatter(sv, [q], flat - r0 * RW)
            return carry

        lax.fori_loop(0, per_w // LANES, build, 0)

        def fire(g, slot, sin, sout):
            pltpu.async_copy(
                ine_hbm.at[ixi.at[pl.ds(g * 2 * G, 2 * G)]],
                rin.at[slot], sin)
            pltpu.async_copy(
                oute_hbm.at[ixo.at[pl.ds(g * 2 * G, 2 * G)]],
                rout.at[slot], sout)

        def wait_slot(slot, sin, sout):
            pltpu.make_async_copy(
                ine_hbm.at[pl.ds(0, 2 * G), :], rin.at[slot], sin).wait()
            pltpu.make_async_copy(
                oute_hbm.at[pl.ds(0, 2 * G), :], rout.at[slot], sout).wait()

        def hsum_all(v):
            for p in perms:
                v = v + jnp.take_along_axis(
                    v, p, axis=0, mode="promise_in_bounds")
            return v

        def compute(g, slot):
            def half(h):
                def item(t, resvec):
                    tl = h * LANES + t
                    q16 = g * G + tl + zero16
                    wi = plsc.load_gather(svi, [q16]) + lane
                    wo = plsc.load_gather(svo, [q16]) + lane

                    def chunk(off, acc):
                        a, b = None, None
                        for w, buf in ((wi, rin), (wo, rout)):
                            wc = w + off
                            cross = (wc >= RW).astype(jnp.int32)
                            v = plsc.load_gather(
                                buf.at[slot],
                                [2 * tl + cross, wc - cross * RW])
                            if a is None:
                                a = v
                            else:
                                b = v
                        return acc + a * b if acc is not None else a * b

                    acc = chunk(0, None)
                    for c in range(1, n_full):
                        acc = chunk(c * LANES, acc)
                    tail = chunk(tail_base, None)
                    acc += jnp.where(tail_mask, tail, jnp.float32(0.0))
                    return jnp.where(lane == t, hsum_all(acc), resvec)

                resvec = lax.fori_loop(
                    0, LANES, item, jnp.zeros((LANES,), jnp.float32))
                res_v[pl.ds(g * G + h * LANES, LANES)] = resvec

            for h in range(G // LANES):
                half(h)

        for s in range(DEPTH):
            fire(s, s, *sems[s])

        def super_body(kk, carry):
            g0 = kk * DEPTH
            for s in range(DEPTH):
                wait_slot(s, *sems[s])
                compute(g0 + s, s)
                fire(g0 + s + DEPTH, s, *sems[s])
            return carry

        lax.fori_loop(0, SUPER - 1, super_body, 0)

        for s in range(DEPTH):
            g = (SUPER - 1) * DEPTH + s
            wait_slot(s, *sems[s])
            compute(g, s)

        pltpu.sync_copy(res_v, out_hbm.at[pl.ds(base, per_w)])

    return k


@jax.jit
def kernel(x, input_embedding, output_embedding):
    B = x.shape[0]
    V, D = input_embedding.shape
    k = _make_kernel(B, V, D)
    return k(x, input_embedding.reshape(-1, RW),
             output_embedding.reshape(-1, RW))


# speedup vs baseline: 1.1853x; 1.1853x over previous
import functools

import jax
import jax.numpy as jnp
from jax import lax
from jax.experimental import pallas as pl
from jax.experimental.pallas import tpu as pltpu
from jax.experimental.pallas import tpu_sc as plsc

NC = 2
NS = 16
NW = NC * NS
LANES = 16
SUB = 8
G = 4
DEPTH = 2


def _make_kernel(B, V, D):
    per_w = B // NW
    NG = per_w // G
    SUPER = NG // DEPTH
    n_full = D // LANES
    rem = D - n_full * LANES
    tail_base = D - LANES

    mesh = plsc.VectorSubcoreMesh(core_axis_name="c", subcore_axis_name="s")

    @functools.partial(
        pl.kernel,
        mesh=mesh,
        compiler_params=pltpu.CompilerParams(
            use_tc_tiling_on_sc=True, needs_layout_passes=False),
        out_type=jax.ShapeDtypeStruct((B,), jnp.float32),
        scratch_types=[
            pltpu.VMEM((per_w, 2), jnp.int32),
            pltpu.VMEM((per_w,), jnp.int32),
            pltpu.VMEM((per_w,), jnp.int32),
            pltpu.VMEM((per_w,), jnp.int32),
            pltpu.VMEM((per_w,), jnp.int32),
            pltpu.VMEM((DEPTH, G, SUB, D), jnp.float32),
            pltpu.VMEM((DEPTH, G, SUB, D), jnp.float32),
            pltpu.VMEM((per_w + LANES - G,), jnp.float32),
            pltpu.SemaphoreType.DMA,
            pltpu.SemaphoreType.DMA,
            pltpu.SemaphoreType.DMA,
            pltpu.SemaphoreType.DMA,
        ],
    )
    def k(x_hbm, ine_hbm, oute_hbm, out_hbm,
          xv, ixi, ixo, rvi, rvo, rin, rout, res_v, si0, so0, si1, so1):
        wid = lax.axis_index("s") * NC + lax.axis_index("c")
        base = wid * per_w
        lane = lax.iota(jnp.int32, LANES)
        zero16 = lane * 0
        one16 = zero16 + 1
        tail_mask = lane >= (LANES - rem)
        perms = [lane ^ kk for kk in (8, 4, 2, 1)]
        sems = [(si0, so0), (si1, so1)]

        pltpu.sync_copy(x_hbm.at[pl.ds(base, per_w), :], xv)

        def build(cc, carry):
            q = cc * LANES + lane
            for col, ix, rv in ((zero16, ixi, rvi), (one16, ixo, rvo)):
                j = plsc.load_gather(xv, [q, col])
                plsc.store_scatter(ix, [q], j // SUB)
                plsc.store_scatter(rv, [q], j - (j // SUB) * SUB)
            return carry

        lax.fori_loop(0, per_w // LANES, build, 0)

        def fire(g, slot, sin, sout):
            mi = plsc.load_gather(ixi, [g * G + lane])
            mo = plsc.load_gather(ixo, [g * G + lane])
            for t in range(G):
                pltpu.async_copy(
                    ine_hbm.at[pl.ds(mi[t], 1)],
                    rin.at[slot, pl.ds(t, 1)], sin)
                pltpu.async_copy(
                    oute_hbm.at[pl.ds(mo[t], 1)],
                    rout.at[slot, pl.ds(t, 1)], sout)

        def wait_slot(slot, sin, sout):
            pltpu.make_async_copy(
                ine_hbm.at[pl.ds(0, G), :, :], rin.at[slot], sin).wait()
            pltpu.make_async_copy(
                oute_hbm.at[pl.ds(0, G), :, :], rout.at[slot], sout).wait()

        def hsum_all(v):
            for p in perms:
                v = v + jnp.take_along_axis(
                    v, p, axis=0, mode="promise_in_bounds")
            return v

        def compute(g, slot):
            def item(t, resvec):
                q16 = g * G + t + zero16
                ri = plsc.load_gather(rvi, [q16])
                ro = plsc.load_gather(rvo, [q16])
                t16 = t + zero16

                def chunk(off, acc):
                    col = off + lane
                    a = plsc.load_gather(rin.at[slot], [t16, ri, col])
                    b = plsc.load_gather(rout.at[slot], [t16, ro, col])
                    return acc + a * b if acc is not None else a * b

                acc = chunk(0, None)
                for c in range(1, n_full):
                    acc = chunk(c * LANES, acc)
                tail = chunk(tail_base, None)
                acc += jnp.where(tail_mask, tail, jnp.float32(0.0))
                return jnp.where(lane == t, hsum_all(acc), resvec)

            resvec = lax.fori_loop(
                0, G, item, jnp.zeros((LANES,), jnp.float32))
            res_v[pl.ds(g * G, LANES)] = resvec

        for s in range(DEPTH):
            fire(s, s, *sems[s])

        def super_body(kk, carry):
            g0 = kk * DEPTH
            for s in range(DEPTH):
                wait_slot(s, *sems[s])
                compute(g0 + s, s)
                fire(g0 + s + DEPTH, s, *sems[s])
            return carry

        lax.fori_loop(0, SUPER - 1, super_body, 0)

        for s in range(DEPTH):
            g = (SUPER - 1) * DEPTH + s
            wait_slot(s, *sems[s])
            compute(g, s)

        pltpu.sync_copy(res_v.at[pl.ds(0, per_w)],
                        out_hbm.at[pl.ds(base, per_w)])

    return k


@jax.jit
def kernel(x, input_embedding, output_embedding):
    B = x.shape[0]
    V, D = input_embedding.shape
    k = _make_kernel(B, V, D)
    return k(x, input_embedding.reshape(V // SUB, SUB, D),
             output_embedding.reshape(V // SUB, SUB, D))


# speedup vs baseline: 4.5919x; 3.8741x over previous
import functools

import jax
import jax.numpy as jnp
from jax import lax
from jax.experimental import pallas as pl
from jax.experimental.pallas import tpu as pltpu
from jax.experimental.pallas import tpu_sc as plsc

NC = 2
NS = 16
NW = NC * NS
LANES = 16
DEPTH = 2


def _make_kernel(B, V, D):
    per_w = B // NW
    NG = per_w // LANES
    SUPER = NG // DEPTH
    n_full = D // LANES
    rem = D - n_full * LANES
    tail_base = D - LANES

    mesh = plsc.VectorSubcoreMesh(core_axis_name="c", subcore_axis_name="s")

    @functools.partial(
        pl.kernel,
        mesh=mesh,
        compiler_params=pltpu.CompilerParams(
            use_tc_tiling_on_sc=True, needs_layout_passes=False),
        out_type=jax.ShapeDtypeStruct((B,), jnp.float32),
        scratch_types=[
            pltpu.VMEM((per_w, 2), jnp.int32),
            pltpu.VMEM((DEPTH, LANES, D), jnp.float32),
            pltpu.VMEM((DEPTH, LANES, D), jnp.float32),
            pltpu.VMEM((per_w,), jnp.float32),
            pltpu.SemaphoreType.DMA,
            pltpu.SemaphoreType.DMA,
            pltpu.SemaphoreType.DMA,
            pltpu.SemaphoreType.DMA,
        ],
    )
    def k(x_hbm, ine_hbm, oute_hbm, out_hbm,
          xv, rin, rout, res_v, si0, so0, si1, so1):
        wid = lax.axis_index("s") * NC + lax.axis_index("c")
        base = wid * per_w
        lane = lax.iota(jnp.int32, LANES)
        zero16 = lane * 0
        one16 = zero16 + 1
        tail_mask = lane >= (LANES - rem)
        perms = [lane ^ kk for kk in (8, 4, 2, 1)]
        sems = [(si0, so0), (si1, so1)]

        pltpu.sync_copy(x_hbm.at[pl.ds(base, per_w), :], xv)

        def fire(g, slot, sin, sout):
            rows = g * LANES + lane
            iv0 = plsc.load_gather(xv, [rows, zero16])
            iv1 = plsc.load_gather(xv, [rows, one16])
            for t in range(LANES):
                pltpu.async_copy(
                    ine_hbm.at[pl.ds(iv0[t], 1), :],
                    rin.at[slot, pl.ds(t, 1), :], sin)
                pltpu.async_copy(
                    oute_hbm.at[pl.ds(iv1[t], 1), :],
                    rout.at[slot, pl.ds(t, 1), :], sout)

        def wait_slot(slot, sin, sout):
            pltpu.make_async_copy(
                ine_hbm.at[pl.ds(0, LANES), :], rin.at[slot], sin).wait()
            pltpu.make_async_copy(
                oute_hbm.at[pl.ds(0, LANES), :], rout.at[slot], sout).wait()

        def hsum_all(v):
            for p in perms:
                v = v + jnp.take_along_axis(
                    v, p, axis=0, mode="promise_in_bounds")
            return v

        def compute(g, slot):
            def item(t, resvec):
                acc = (rin[slot, t, pl.ds(0, LANES)]
                       * rout[slot, t, pl.ds(0, LANES)])
                for c in range(1, n_full):
                    acc += (rin[slot, t, pl.ds(c * LANES, LANES)]
                            * rout[slot, t, pl.ds(c * LANES, LANES)])
                tail = (rin[slot, t, pl.ds(tail_base, LANES)]
                        * rout[slot, t, pl.ds(tail_base, LANES)])
                acc += jnp.where(tail_mask, tail, jnp.float32(0.0))
                return jnp.where(lane == t, hsum_all(acc), resvec)

            resvec = lax.fori_loop(
                0, LANES, item, jnp.zeros((LANES,), jnp.float32))
            res_v[pl.ds(g * LANES, LANES)] = resvec

        for s in range(DEPTH):
            fire(s, s, *sems[s])

        def super_body(kk, carry):
            g0 = kk * DEPTH
            for s in range(DEPTH):
                wait_slot(s, *sems[s])
                compute(g0 + s, s)
                fire(g0 + s + DEPTH, s, *sems[s])
            return carry

        lax.fori_loop(0, SUPER - 1, super_body, 0)

        for s in range(DEPTH):
            g = (SUPER - 1) * DEPTH + s
            wait_slot(s, *sems[s])
            compute(g, s)

        pltpu.sync_copy(res_v, out_hbm.at[pl.ds(base, per_w)])

    return k


@jax.jit
def kernel(x, input_embedding, output_embedding):
    B = x.shape[0]
    V, D = input_embedding.shape
    k = _make_kernel(B, V, D)
    return k(x, input_embedding, output_embedding)
